# baseline (speedup 1.0000x reference)
import functools

import jax
import jax.numpy as jnp
from jax import lax
from jax.experimental import pallas as pl
from jax.experimental.pallas import tpu as pltpu
from jax.experimental.pallas import tpu_sc as plsc

N = 100000
E = 1600000
D = 32

NC, NS = 2, 16

TPH = E // 8
H_CH = 4096
H_NF = 48
H_TAIL = TPH - H_NF * H_CH
HR = 6400
HSL = HR // 8

TPE = E // NS
HALF = N // 2
TRASH = 128
AR = 50176
R = 256
M_NF = TPE // R
M_STEPS = M_NF // 2
M_TAIL = TPE - M_NF * R
ZB_R = 196

_MESH = plsc.VectorSubcoreMesh(core_axis_name="c", subcore_axis_name="s",
                               num_cores=NC, num_subcores=NS)
_SC_PARAMS = pltpu.CompilerParams(needs_layout_passes=False,
                                  use_tc_tiling_on_sc=False)


@functools.partial(
    pl.kernel,
    out_type=jax.ShapeDtypeStruct((4 * HR, 16), jnp.float32),
    mesh=_MESH,
    compiler_params=_SC_PARAMS,
    scratch_types=[
        pltpu.VMEM_SHARED((2 * HR, 16), jnp.float32),
        pltpu.VMEM((HR, 16), jnp.float32),
        pltpu.VMEM((2, H_CH), jnp.int32),
        pltpu.VMEM((HR,), jnp.int32),
        pltpu.SemaphoreType.DMA,
        pltpu.SemaphoreType.DMA,
    ],
)
def _hist_kernel(eall_hbm, rowids_hbm, deg_hbm, acc_sh, part, ibuf, rid, sem0, sem1):
    c = lax.axis_index("c")
    t = lax.axis_index("s")
    which = t // 8
    g = t % 8
    pltpu.sync_copy(rowids_hbm.at[pl.ds(which * HR, HR)], rid)
    z16 = jnp.zeros((16,), jnp.float32)
    ones = jnp.ones((16,), jnp.float32)
    sems = (sem0, sem1)

    @pl.loop(0, HR)
    def _(i):
        part[i, :] = z16

    acc_base = which * HR + g * HSL
    pltpu.sync_copy(part.at[pl.ds(0, HSL), :],
                    acc_sh.at[pl.ds(acc_base, HSL), :])
    plsc.subcore_barrier()

    base = (2 * c + which) * E + g * TPH

    def _count(n_vregs, d):
        @pl.loop(0, n_vregs)
        def _(r):
            v = ibuf[d, pl.ds(r * 16, 16)]
            row = jax.lax.shift_right_logical(v, 4)
            col = jax.lax.bitwise_and(v, 15)
            plsc.addupdate_scatter(part, [row, col], ones)

    for d in range(2):
        pltpu.async_copy(eall_hbm.at[pl.ds(base + d * H_CH, H_CH)],
                         ibuf.at[d], sems[d])

    @pl.loop(0, H_NF // 2)
    def _(m):
        for d in range(2):
            k = 2 * m + d
            pltpu.make_async_copy(
                eall_hbm.at[pl.ds(base + k * H_CH, H_CH)], ibuf.at[d],
                sems[d]).wait()
            _count(H_CH // 16, d)

            @pl.when(m < H_NF // 2 - 1)
            def _():
                nxt = base + (k + 2) * H_CH
                pltpu.async_copy(eall_hbm.at[pl.ds(nxt, H_CH)],
                                 ibuf.at[d], sems[d])

    tail_off = base + H_NF * H_CH
    pltpu.async_copy(eall_hbm.at[pl.ds(tail_off, H_TAIL)],
                     ibuf.at[0, pl.ds(0, H_TAIL)], sem0)
    pltpu.make_async_copy(eall_hbm.at[pl.ds(tail_off, H_TAIL)],
                          ibuf.at[0, pl.ds(0, H_TAIL)], sem0).wait()
    _count(H_TAIL // 16, 0)

    pltpu.sync_copy(part, acc_sh.at[rid], add=True)

    plsc.subcore_barrier()
    out_base = (2 * c + which) * HR + g * HSL
    pltpu.sync_copy(acc_sh.at[pl.ds(acc_base, HSL), :],
                    deg_hbm.at[pl.ds(out_base, HSL), :])


@functools.partial(
    pl.kernel,
    out_type=(jax.ShapeDtypeStruct((N, D), jnp.float32),
              jax.ShapeDtypeStruct((N, D), jnp.float32)),
    mesh=_MESH,
    compiler_params=_SC_PARAMS,
    scratch_types=[
        pltpu.VMEM_SHARED((AR, D), jnp.float32),
        pltpu.VMEM((R,), jnp.int32),
        pltpu.VMEM((R,), jnp.int32),
        pltpu.VMEM((R,), jnp.int32),
        pltpu.VMEM((R,), jnp.int32),
        pltpu.VMEM((R,), jnp.int32),
        pltpu.VMEM((R,), jnp.int32),
        pltpu.VMEM((R, D), jnp.float32),
        pltpu.VMEM((R, D), jnp.float32),
        pltpu.VMEM((M_TAIL,), jnp.int32),
        pltpu.VMEM((M_TAIL,), jnp.int32),
        pltpu.VMEM((M_TAIL,), jnp.int32),
        pltpu.VMEM((M_TAIL, D), jnp.float32),
        pltpu.VMEM((ZB_R, D), jnp.float32),
        pltpu.SemaphoreType.DMA,
        pltpu.SemaphoreType.DMA,
        pltpu.SemaphoreType.DMA,
        pltpu.SemaphoreType.DMA,
        pltpu.SemaphoreType.DMA,
        pltpu.SemaphoreType.DMA,
        pltpu.SemaphoreType.DMA,
    ],
)
def _agg_kernel(z1_hbm, z2_hbm, eall_hbm, acc1_hbm, acc2_hbm, acc_sh,
                sb0, sb1, tb0, tb1, lb0, lb1, rw0, rw1, sbt, tbt, lbt, rwt,
                zb, si0, si1, sg0, sg1, ss0, ss1, sz):
    c = lax.axis_index("c")
    t = lax.axis_index("s")
    base_row = c * HALF
    z16 = jnp.zeros((16,), jnp.float32)

    @pl.loop(0, ZB_R)
    def _(i):
        zb[i, pl.ds(0, 16)] = z16
        zb[i, pl.ds(16, 16)] = z16

    tile_base = t * TPE
    sb = (sb0, sb1)
    tb = (tb0, tb1)
    lb = (lb0, lb1)
    rw = (rw0, rw1)
    sem_i = (si0, si1)
    sem_g = (sg0, sg1)
    sem_s = (ss0, ss1)

    def _remap(tref, lref, n_vregs):
        @pl.loop(0, n_vregs)
        def _(i):
            v = tref[pl.ds(i * 16, 16)]
            tl = v - base_row
            ok = jnp.logical_and(tl >= 0, tl < HALF)
            trash = HALF + jax.lax.bitwise_and(v, TRASH - 1)
            lref[pl.ds(i * 16, 16)] = jnp.where(ok, tl, trash)

    for rel in range(2):
        z_hbm = (z1_hbm, z2_hbm)[rel]
        acc_hbm = (acc1_hbm, acc2_hbm)[rel]
        e_src = 2 * rel * E
        e_dst = (2 * rel + 1) * E

        for i in range((AR // NS) // ZB_R):
            pltpu.async_copy(
                zb, acc_sh.at[pl.ds(t * (AR // NS) + i * ZB_R, ZB_R), :], sz)
        for i in range((AR // NS) // ZB_R):
            pltpu.make_async_copy(
                zb, acc_sh.at[pl.ds(t * (AR // NS) + i * ZB_R, ZB_R), :], sz).wait()
        plsc.subcore_barrier()

        for d in range(2):
            pltpu.async_copy(eall_hbm.at[pl.ds(e_src + tile_base + d * R, R)],
                             sb[d], sem_i[d])
            pltpu.async_copy(eall_hbm.at[pl.ds(e_dst + tile_base + d * R, R)],
                             tb[d], sem_i[d])

        @pl.loop(0, M_STEPS)
        def _(m):
            for d in range(2):
                k = 2 * m + d
                off = tile_base + k * R
                pltpu.make_async_copy(eall_hbm.at[pl.ds(e_src + off, R)],
                                      sb[d], sem_i[d]).wait()
                pltpu.make_async_copy(eall_hbm.at[pl.ds(e_dst + off, R)],
                                      tb[d], sem_i[d]).wait()

                @pl.when(m > 0)
                def _():
                    pltpu.make_async_copy(rw[d], acc_sh.at[lb[d]],
                                          sem_s[d]).wait()

                pltpu.async_copy(z_hbm.at[sb[d]], rw[d], sem_g[d])
                _remap(tb[d], lb[d], R // 16)
                pltpu.make_async_copy(z_hbm.at[sb[d]], rw[d], sem_g[d]).wait()
                pltpu.async_copy(rw[d], acc_sh.at[lb[d]], sem_s[d], add=True)

                @pl.when(m < M_STEPS - 1)
                def _():
                    nxt = tile_base + (k + 2) * R
                    pltpu.async_copy(eall_hbm.at[pl.ds(e_src + nxt, R)],
                                     sb[d], sem_i[d])
                    pltpu.async_copy(eall_hbm.at[pl.ds(e_dst + nxt, R)],
                                     tb[d], sem_i[d])

        for d in range(2):
            pltpu.make_async_copy(rw[d], acc_sh.at[lb[d]], sem_s[d]).wait()

        toff = tile_base + M_NF * R
        pltpu.async_copy(eall_hbm.at[pl.ds(e_src + toff, M_TAIL)], sbt, si0)
        pltpu.async_copy(eall_hbm.at[pl.ds(e_dst + toff, M_TAIL)], tbt, si0)
        pltpu.make_async_copy(eall_hbm.at[pl.ds(e_src + toff, M_TAIL)], sbt,
                              si0).wait()
        pltpu.make_async_copy(eall_hbm.at[pl.ds(e_dst + toff, M_TAIL)], tbt,
                              si0).wait()
        pltpu.async_copy(z_hbm.at[sbt], rwt, sg0)
        _remap(tbt, lbt, M_TAIL // 16)
        pltpu.make_async_copy(z_hbm.at[sbt], rwt, sg0).wait()
        pltpu.async_copy(rwt, acc_sh.at[lbt], ss0, add=True)
        pltpu.make_async_copy(rwt, acc_sh.at[lbt], ss0).wait()

        plsc.subcore_barrier()
        rows_per_tile = HALF // NS
        pltpu.sync_copy(
            acc_sh.at[pl.ds(t * rows_per_tile, rows_per_tile), :],
            acc_hbm.at[pl.ds(base_row + t * rows_per_tile, rows_per_tile), :])
        plsc.subcore_barrier()


_TCB = 5000


def _scale_matmul(h, deg, w):
    def body(h_ref, d_ref, w_ref, z_ref):
        sc = jax.lax.rsqrt(jnp.maximum(d_ref[...], 1.0))
        z_ref[...] = jnp.dot(h_ref[...] * sc, w_ref[...],
                             preferred_element_type=jnp.float32,
                             precision=jax.lax.Precision.HIGHEST)

    return pl.pallas_call(
        body,
        out_shape=jax.ShapeDtypeStruct((N, D), jnp.float32),
        grid=(N // _TCB,),
        in_specs=[pl.BlockSpec((_TCB, D), lambda i: (i, 0)),
                  pl.BlockSpec((_TCB, 1), lambda i: (i, 0)),
                  pl.BlockSpec((D, D), lambda i: (0, 0))],
        out_specs=pl.BlockSpec((_TCB, D), lambda i: (i, 0)),
    )(h, deg, w)


def _finalize(acc, deg, b):
    def body(a_ref, d_ref, b_ref, o_ref):
        sc = jax.lax.rsqrt(jnp.maximum(d_ref[...], 1.0))
        y = a_ref[...] * sc + b_ref[...]
        o_ref[...] = jnp.where(y > 0, y, jnp.exp(jnp.minimum(y, 0.0)) - 1.0)

    return pl.pallas_call(
        body,
        out_shape=jax.ShapeDtypeStruct((N, D), jnp.float32),
        grid=(N // _TCB,),
        in_specs=[pl.BlockSpec((_TCB, D), lambda i: (i, 0)),
                  pl.BlockSpec((_TCB, 1), lambda i: (i, 0)),
                  pl.BlockSpec((1, D), lambda i: (0, 0))],
        out_specs=pl.BlockSpec((_TCB, D), lambda i: (i, 0)),
    )(acc, deg, b)


def kernel(h_user, h_item, edge_index_user_to_item, edge_index_item_to_user, W, b):
    rowids = jnp.arange(2 * HR, dtype=jnp.int32)
    eall = jnp.concatenate([edge_index_user_to_item.reshape(-1),
                            edge_index_item_to_user.reshape(-1)])

    deg = _hist_kernel(eall, rowids)
    degf = deg.reshape(4, HR * 16)[:, :N]
    dout1, din1, dout2, din2 = (degf[i].reshape(N, 1) for i in range(4))

    z1 = _scale_matmul(h_user, dout1, W)
    z2 = _scale_matmul(h_item, dout2, W)
    acc1, acc2 = _agg_kernel(z1, z2, eall)
    out_item = _finalize(acc1, din1, b.reshape(1, D))
    out_user = _finalize(acc2, din2, b.reshape(1, D))
    return (out_user, out_item)


# speedup vs baseline: 1.1182x; 1.1182x over previous
import functools

import jax
import jax.numpy as jnp
from jax import lax
from jax.experimental import pallas as pl
from jax.experimental.pallas import tpu as pltpu
from jax.experimental.pallas import tpu_sc as plsc

N = 100000
E = 1600000
D = 32

NC, NS = 2, 16

TPH = E // 8
H_CH = 4096
H_NF = 48
H_TAIL = TPH - H_NF * H_CH
HR = 6400
HSL = HR // 8

TPE = E // NS
HALF = N // 2
TRASH = 128
AR = 50176
R = 256
M_NF = TPE // R
M_STEPS = M_NF // 2
M_TAIL = TPE - M_NF * R
ZB_R = 196

_MESH = plsc.VectorSubcoreMesh(core_axis_name="c", subcore_axis_name="s",
                               num_cores=NC, num_subcores=NS)
_SC_PARAMS = pltpu.CompilerParams(needs_layout_passes=False,
                                  use_tc_tiling_on_sc=False)


@functools.partial(
    pl.kernel,
    out_type=jax.ShapeDtypeStruct((4 * HR, 16), jnp.float32),
    mesh=_MESH,
    compiler_params=_SC_PARAMS,
    scratch_types=[
        pltpu.VMEM_SHARED((2 * HR, 16), jnp.float32),
        pltpu.VMEM((HR, 16), jnp.float32),
        pltpu.VMEM((2, H_CH), jnp.int32),
        pltpu.VMEM((HR,), jnp.int32),
        pltpu.SemaphoreType.DMA,
        pltpu.SemaphoreType.DMA,
    ],
)
def _hist_kernel(eall_hbm, rowids_hbm, deg_hbm, acc_sh, part, ibuf, rid, sem0, sem1):
    c = lax.axis_index("c")
    t = lax.axis_index("s")
    which = t // 8
    g = t % 8
    pltpu.sync_copy(rowids_hbm.at[pl.ds(which * HR, HR)], rid)
    z16 = jnp.zeros((16,), jnp.float32)
    ones = jnp.ones((16,), jnp.float32)
    sems = (sem0, sem1)

    @pl.loop(0, HR)
    def _(i):
        part[i, :] = z16

    acc_base = which * HR + g * HSL
    pltpu.sync_copy(part.at[pl.ds(0, HSL), :],
                    acc_sh.at[pl.ds(acc_base, HSL), :])
    plsc.subcore_barrier()

    base = (2 * c + which) * E + g * TPH

    def _count(n_vregs, d):
        @pl.loop(0, n_vregs)
        def _(r):
            v = ibuf[d, pl.ds(r * 16, 16)]
            row = jax.lax.shift_right_logical(v, 4)
            col = jax.lax.bitwise_and(v, 15)
            plsc.addupdate_scatter(part, [row, col], ones)

    for d in range(2):
        pltpu.async_copy(eall_hbm.at[pl.ds(base + d * H_CH, H_CH)],
                         ibuf.at[d], sems[d])

    @pl.loop(0, H_NF // 2)
    def _(m):
        for d in range(2):
            k = 2 * m + d
            pltpu.make_async_copy(
                eall_hbm.at[pl.ds(base + k * H_CH, H_CH)], ibuf.at[d],
                sems[d]).wait()
            _count(H_CH // 16, d)

            @pl.when(m < H_NF // 2 - 1)
            def _():
                nxt = base + (k + 2) * H_CH
                pltpu.async_copy(eall_hbm.at[pl.ds(nxt, H_CH)],
                                 ibuf.at[d], sems[d])

    tail_off = base + H_NF * H_CH
    pltpu.async_copy(eall_hbm.at[pl.ds(tail_off, H_TAIL)],
                     ibuf.at[0, pl.ds(0, H_TAIL)], sem0)
    pltpu.make_async_copy(eall_hbm.at[pl.ds(tail_off, H_TAIL)],
                          ibuf.at[0, pl.ds(0, H_TAIL)], sem0).wait()
    _count(H_TAIL // 16, 0)

    pltpu.sync_copy(part, acc_sh.at[rid], add=True)

    plsc.subcore_barrier()
    out_base = (2 * c + which) * HR + g * HSL
    pltpu.sync_copy(acc_sh.at[pl.ds(acc_base, HSL), :],
                    deg_hbm.at[pl.ds(out_base, HSL), :])


@functools.partial(
    pl.kernel,
    out_type=jax.ShapeDtypeStruct((N, D), jnp.float32),
    mesh=_MESH,
    compiler_params=_SC_PARAMS,
    scratch_types=[
        pltpu.VMEM_SHARED((AR, D), jnp.float32),
        pltpu.VMEM((R,), jnp.int32),
        pltpu.VMEM((R,), jnp.int32),
        pltpu.VMEM((R,), jnp.int32),
        pltpu.VMEM((R,), jnp.int32),
        pltpu.VMEM((R,), jnp.int32),
        pltpu.VMEM((R,), jnp.int32),
        pltpu.VMEM((R, D), jnp.float32),
        pltpu.VMEM((R, D), jnp.float32),
        pltpu.VMEM((M_TAIL,), jnp.int32),
        pltpu.VMEM((M_TAIL,), jnp.int32),
        pltpu.VMEM((M_TAIL,), jnp.int32),
        pltpu.VMEM((M_TAIL, D), jnp.float32),
        pltpu.VMEM((ZB_R, D), jnp.float32),
        pltpu.SemaphoreType.DMA,
        pltpu.SemaphoreType.DMA,
        pltpu.SemaphoreType.DMA,
        pltpu.SemaphoreType.DMA,
        pltpu.SemaphoreType.DMA,
        pltpu.SemaphoreType.DMA,
        pltpu.SemaphoreType.DMA,
    ],
)
def _agg_kernel(z_hbm, e_hbm, acc_hbm, acc_sh, sb0, sb1, tb0, tb1,
                lb0, lb1, rw0, rw1, sbt, tbt, lbt, rwt, zb,
                si0, si1, sg0, sg1, ss0, ss1, sz):
    c = lax.axis_index("c")
    t = lax.axis_index("s")
    base_row = c * HALF
    z16 = jnp.zeros((16,), jnp.float32)

    @pl.loop(0, ZB_R)
    def _(i):
        zb[i, pl.ds(0, 16)] = z16
        zb[i, pl.ds(16, 16)] = z16

    for i in range((AR // NS) // ZB_R):
        pltpu.async_copy(
            zb, acc_sh.at[pl.ds(t * (AR // NS) + i * ZB_R, ZB_R), :], sz)
    for i in range((AR // NS) // ZB_R):
        pltpu.make_async_copy(
            zb, acc_sh.at[pl.ds(t * (AR // NS) + i * ZB_R, ZB_R), :], sz).wait()

    plsc.subcore_barrier()

    tile_base = t * TPE
    sb = (sb0, sb1)
    tb = (tb0, tb1)
    lb = (lb0, lb1)
    rw = (rw0, rw1)
    sem_i = (si0, si1)
    sem_g = (sg0, sg1)
    sem_s = (ss0, ss1)

    def _remap(tref, lref, n_vregs):
        @pl.loop(0, n_vregs)
        def _(i):
            v = tref[pl.ds(i * 16, 16)]
            tl = v - base_row
            ok = jnp.logical_and(tl >= 0, tl < HALF)
            trash = HALF + jax.lax.bitwise_and(v, TRASH - 1)
            lref[pl.ds(i * 16, 16)] = jnp.where(ok, tl, trash)

    for d in range(2):
        pltpu.async_copy(e_hbm.at[0, pl.ds(tile_base + d * R, R)], sb[d], sem_i[d])
        pltpu.async_copy(e_hbm.at[1, pl.ds(tile_base + d * R, R)], tb[d], sem_i[d])

    @pl.loop(0, M_STEPS)
    def _(m):
        for d in range(2):
            k = 2 * m + d
            off = tile_base + k * R
            pltpu.make_async_copy(e_hbm.at[0, pl.ds(off, R)], sb[d],
                                  sem_i[d]).wait()
            pltpu.make_async_copy(e_hbm.at[1, pl.ds(off, R)], tb[d],
                                  sem_i[d]).wait()

            @pl.when(m > 0)
            def _():
                pltpu.make_async_copy(rw[d], acc_sh.at[lb[d]],
                                      sem_s[d]).wait()

            pltpu.async_copy(z_hbm.at[sb[d]], rw[d], sem_g[d])
            _remap(tb[d], lb[d], R // 16)
            pltpu.make_async_copy(z_hbm.at[sb[d]], rw[d], sem_g[d]).wait()
            pltpu.async_copy(rw[d], acc_sh.at[lb[d]], sem_s[d], add=True)

            @pl.when(m < M_STEPS - 1)
            def _():
                nxt = tile_base + (k + 2) * R
                pltpu.async_copy(e_hbm.at[0, pl.ds(nxt, R)], sb[d], sem_i[d])
                pltpu.async_copy(e_hbm.at[1, pl.ds(nxt, R)], tb[d], sem_i[d])

    for d in range(2):
        pltpu.make_async_copy(rw[d], acc_sh.at[lb[d]], sem_s[d]).wait()

    toff = tile_base + M_NF * R
    pltpu.async_copy(e_hbm.at[0, pl.ds(toff, M_TAIL)], sbt, si0)
    pltpu.async_copy(e_hbm.at[1, pl.ds(toff, M_TAIL)], tbt, si0)
    pltpu.make_async_copy(e_hbm.at[0, pl.ds(toff, M_TAIL)], sbt, si0).wait()
    pltpu.make_async_copy(e_hbm.at[1, pl.ds(toff, M_TAIL)], tbt, si0).wait()
    pltpu.async_copy(z_hbm.at[sbt], rwt, sg0)
    _remap(tbt, lbt, M_TAIL // 16)
    pltpu.make_async_copy(z_hbm.at[sbt], rwt, sg0).wait()
    pltpu.async_copy(rwt, acc_sh.at[lbt], ss0, add=True)
    pltpu.make_async_copy(rwt, acc_sh.at[lbt], ss0).wait()

    plsc.subcore_barrier()
    rows_per_tile = HALF // NS
    pltpu.sync_copy(
        acc_sh.at[pl.ds(t * rows_per_tile, rows_per_tile), :],
        acc_hbm.at[pl.ds(base_row + t * rows_per_tile, rows_per_tile), :])


_TCB = 5000


def _scale_matmul(h, deg, w):
    def body(h_ref, d_ref, w_ref, z_ref):
        sc = jax.lax.rsqrt(jnp.maximum(d_ref[...], 1.0))
        z_ref[...] = jnp.dot(h_ref[...] * sc, w_ref[...],
                             preferred_element_type=jnp.float32,
                             precision=jax.lax.Precision.HIGHEST)

    return pl.pallas_call(
        body,
        out_shape=jax.ShapeDtypeStruct((N, D), jnp.float32),
        grid=(N // _TCB,),
        in_specs=[pl.BlockSpec((_TCB, D), lambda i: (i, 0)),
                  pl.BlockSpec((_TCB, 1), lambda i: (i, 0)),
                  pl.BlockSpec((D, D), lambda i: (0, 0))],
        out_specs=pl.BlockSpec((_TCB, D), lambda i: (i, 0)),
    )(h, deg, w)


def _finalize(acc, deg, b):
    def body(a_ref, d_ref, b_ref, o_ref):
        sc = jax.lax.rsqrt(jnp.maximum(d_ref[...], 1.0))
        y = a_ref[...] * sc + b_ref[...]
        o_ref[...] = jnp.where(y > 0, y, jnp.exp(jnp.minimum(y, 0.0)) - 1.0)

    return pl.pallas_call(
        body,
        out_shape=jax.ShapeDtypeStruct((N, D), jnp.float32),
        grid=(N // _TCB,),
        in_specs=[pl.BlockSpec((_TCB, D), lambda i: (i, 0)),
                  pl.BlockSpec((_TCB, 1), lambda i: (i, 0)),
                  pl.BlockSpec((1, D), lambda i: (0, 0))],
        out_specs=pl.BlockSpec((_TCB, D), lambda i: (i, 0)),
    )(acc, deg, b)


def kernel(h_user, h_item, edge_index_user_to_item, edge_index_item_to_user, W, b):
    rowids = jnp.arange(2 * HR, dtype=jnp.int32)
    eall = jnp.concatenate([edge_index_user_to_item.reshape(-1),
                            edge_index_item_to_user.reshape(-1)])

    deg = _hist_kernel(eall, rowids)
    degf = deg.reshape(4, HR * 16)[:, :N]
    dout1, din1, dout2, din2 = (degf[i].reshape(N, 1) for i in range(4))

    z1 = _scale_matmul(h_user, dout1, W)
    z2 = _scale_matmul(h_item, dout2, W)

    acc1 = _agg_kernel(z1, edge_index_user_to_item)
    acc2 = _agg_kernel(z2, edge_index_item_to_user)

    out_item = _finalize(acc1, din1, b.reshape(1, D))
    out_user = _finalize(acc2, din2, b.reshape(1, D))
    return (out_user, out_item)


# speedup vs baseline: 1.1188x; 1.0005x over previous
import functools

import jax
import jax.numpy as jnp
from jax import lax
from jax.experimental import pallas as pl
from jax.experimental.pallas import tpu as pltpu
from jax.experimental.pallas import tpu_sc as plsc

N = 100000
E = 1600000
D = 32

NC, NS = 2, 16

TPH = E // 8
H_CH = 4096
H_NF = 48
H_TAIL = TPH - H_NF * H_CH
HR = 6400
HSL = HR // 8

TPE = E // NS
HALF = N // 2
TRASH = 1024
AR = 51200
R = 256
M_NF = TPE // R
M_STEPS = M_NF // 2
M_TAIL = TPE - M_NF * R
ZB_R = 100

_MESH = plsc.VectorSubcoreMesh(core_axis_name="c", subcore_axis_name="s",
                               num_cores=NC, num_subcores=NS)
_SC_PARAMS = pltpu.CompilerParams(needs_layout_passes=False,
                                  use_tc_tiling_on_sc=False)


@functools.partial(
    pl.kernel,
    out_type=jax.ShapeDtypeStruct((4 * HR, 16), jnp.float32),
    mesh=_MESH,
    compiler_params=_SC_PARAMS,
    scratch_types=[
        pltpu.VMEM_SHARED((2 * HR, 16), jnp.float32),
        pltpu.VMEM((HR, 16), jnp.float32),
        pltpu.VMEM((2, H_CH), jnp.int32),
        pltpu.VMEM((HR,), jnp.int32),
        pltpu.SemaphoreType.DMA,
        pltpu.SemaphoreType.DMA,
    ],
)
def _hist_kernel(eall_hbm, rowids_hbm, deg_hbm, acc_sh, part, ibuf, rid, sem0, sem1):
    c = lax.axis_index("c")
    t = lax.axis_index("s")
    which = t // 8
    g = t % 8
    pltpu.sync_copy(rowids_hbm.at[pl.ds(which * HR, HR)], rid)
    z16 = jnp.zeros((16,), jnp.float32)
    ones = jnp.ones((16,), jnp.float32)
    sems = (sem0, sem1)

    @pl.loop(0, HR)
    def _(i):
        part[i, :] = z16

    acc_base = which * HR + g * HSL
    pltpu.sync_copy(part.at[pl.ds(0, HSL), :],
                    acc_sh.at[pl.ds(acc_base, HSL), :])
    plsc.subcore_barrier()

    base = (2 * c + which) * E + g * TPH

    def _count(n_vregs, d):
        @pl.loop(0, n_vregs)
        def _(r):
            v = ibuf[d, pl.ds(r * 16, 16)]
            row = jax.lax.shift_right_logical(v, 4)
            col = jax.lax.bitwise_and(v, 15)
            plsc.addupdate_scatter(part, [row, col], ones)

    for d in range(2):
        pltpu.async_copy(eall_hbm.at[pl.ds(base + d * H_CH, H_CH)],
                         ibuf.at[d], sems[d])

    @pl.loop(0, H_NF // 2)
    def _(m):
        for d in range(2):
            k = 2 * m + d
            pltpu.make_async_copy(
                eall_hbm.at[pl.ds(base + k * H_CH, H_CH)], ibuf.at[d],
                sems[d]).wait()
            _count(H_CH // 16, d)

            @pl.when(m < H_NF // 2 - 1)
            def _():
                nxt = base + (k + 2) * H_CH
                pltpu.async_copy(eall_hbm.at[pl.ds(nxt, H_CH)],
                                 ibuf.at[d], sems[d])

    tail_off = base + H_NF * H_CH
    pltpu.async_copy(eall_hbm.at[pl.ds(tail_off, H_TAIL)],
                     ibuf.at[0, pl.ds(0, H_TAIL)], sem0)
    pltpu.make_async_copy(eall_hbm.at[pl.ds(tail_off, H_TAIL)],
                          ibuf.at[0, pl.ds(0, H_TAIL)], sem0).wait()
    _count(H_TAIL // 16, 0)

    pltpu.sync_copy(part, acc_sh.at[rid], add=True)

    plsc.subcore_barrier()
    out_base = (2 * c + which) * HR + g * HSL
    pltpu.sync_copy(acc_sh.at[pl.ds(acc_base, HSL), :],
                    deg_hbm.at[pl.ds(out_base, HSL), :])


@functools.partial(
    pl.kernel,
    out_type=jax.ShapeDtypeStruct((N, D), jnp.float32),
    mesh=_MESH,
    compiler_params=_SC_PARAMS,
    scratch_types=[
        pltpu.VMEM_SHARED((AR, D), jnp.float32),
        pltpu.VMEM((R,), jnp.int32),
        pltpu.VMEM((R,), jnp.int32),
        pltpu.VMEM((R,), jnp.int32),
        pltpu.VMEM((R,), jnp.int32),
        pltpu.VMEM((R,), jnp.int32),
        pltpu.VMEM((R,), jnp.int32),
        pltpu.VMEM((R, D), jnp.float32),
        pltpu.VMEM((R, D), jnp.float32),
        pltpu.VMEM((M_TAIL,), jnp.int32),
        pltpu.VMEM((M_TAIL,), jnp.int32),
        pltpu.VMEM((M_TAIL,), jnp.int32),
        pltpu.VMEM((M_TAIL, D), jnp.float32),
        pltpu.VMEM((ZB_R, D), jnp.float32),
        pltpu.SemaphoreType.DMA,
        pltpu.SemaphoreType.DMA,
        pltpu.SemaphoreType.DMA,
        pltpu.SemaphoreType.DMA,
        pltpu.SemaphoreType.DMA,
        pltpu.SemaphoreType.DMA,
        pltpu.SemaphoreType.DMA,
    ],
)
def _agg_kernel(z_hbm, e_hbm, acc_hbm, acc_sh, sb0, sb1, tb0, tb1,
                lb0, lb1, rw0, rw1, sbt, tbt, lbt, rwt, zb,
                si0, si1, sg0, sg1, ss0, ss1, sz):
    c = lax.axis_index("c")
    t = lax.axis_index("s")
    base_row = c * HALF
    z16 = jnp.zeros((16,), jnp.float32)

    @pl.loop(0, ZB_R)
    def _(i):
        zb[i, pl.ds(0, 16)] = z16
        zb[i, pl.ds(16, 16)] = z16

    for i in range((AR // NS) // ZB_R):
        pltpu.async_copy(
            zb, acc_sh.at[pl.ds(t * (AR // NS) + i * ZB_R, ZB_R), :], sz)
    for i in range((AR // NS) // ZB_R):
        pltpu.make_async_copy(
            zb, acc_sh.at[pl.ds(t * (AR // NS) + i * ZB_R, ZB_R), :], sz).wait()

    plsc.subcore_barrier()

    tile_base = t * TPE
    sb = (sb0, sb1)
    tb = (tb0, tb1)
    lb = (lb0, lb1)
    rw = (rw0, rw1)
    sem_i = (si0, si1)
    sem_g = (sg0, sg1)
    sem_s = (ss0, ss1)

    def _remap(tref, lref, n_vregs):
        @pl.loop(0, n_vregs)
        def _(i):
            v = tref[pl.ds(i * 16, 16)]
            tl = v - base_row
            ok = jnp.logical_and(tl >= 0, tl < HALF)
            trash = HALF + jax.lax.bitwise_and(v, TRASH - 1)
            lref[pl.ds(i * 16, 16)] = jnp.where(ok, tl, trash)

    for d in range(2):
        pltpu.async_copy(e_hbm.at[0, pl.ds(tile_base + d * R, R)], sb[d], sem_i[d])
        pltpu.async_copy(e_hbm.at[1, pl.ds(tile_base + d * R, R)], tb[d], sem_i[d])

    @pl.loop(0, M_STEPS)
    def _(m):
        for d in range(2):
            k = 2 * m + d
            off = tile_base + k * R
            pltpu.make_async_copy(e_hbm.at[0, pl.ds(off, R)], sb[d],
                                  sem_i[d]).wait()
            pltpu.make_async_copy(e_hbm.at[1, pl.ds(off, R)], tb[d],
                                  sem_i[d]).wait()

            @pl.when(m > 0)
            def _():
                pltpu.make_async_copy(rw[d], acc_sh.at[lb[d]],
                                      sem_s[d]).wait()

            pltpu.async_copy(z_hbm.at[sb[d]], rw[d], sem_g[d])
            _remap(tb[d], lb[d], R // 16)
            pltpu.make_async_copy(z_hbm.at[sb[d]], rw[d], sem_g[d]).wait()
            pltpu.async_copy(rw[d], acc_sh.at[lb[d]], sem_s[d], add=True)

            @pl.when(m < M_STEPS - 1)
            def _():
                nxt = tile_base + (k + 2) * R
                pltpu.async_copy(e_hbm.at[0, pl.ds(nxt, R)], sb[d], sem_i[d])
                pltpu.async_copy(e_hbm.at[1, pl.ds(nxt, R)], tb[d], sem_i[d])

    for d in range(2):
        pltpu.make_async_copy(rw[d], acc_sh.at[lb[d]], sem_s[d]).wait()

    toff = tile_base + M_NF * R
    pltpu.async_copy(e_hbm.at[0, pl.ds(toff, M_TAIL)], sbt, si0)
    pltpu.async_copy(e_hbm.at[1, pl.ds(toff, M_TAIL)], tbt, si0)
    pltpu.make_async_copy(e_hbm.at[0, pl.ds(toff, M_TAIL)], sbt, si0).wait()
    pltpu.make_async_copy(e_hbm.at[1, pl.ds(toff, M_TAIL)], tbt, si0).wait()
    pltpu.async_copy(z_hbm.at[sbt], rwt, sg0)
    _remap(tbt, lbt, M_TAIL // 16)
    pltpu.make_async_copy(z_hbm.at[sbt], rwt, sg0).wait()
    pltpu.async_copy(rwt, acc_sh.at[lbt], ss0, add=True)
    pltpu.make_async_copy(rwt, acc_sh.at[lbt], ss0).wait()

    plsc.subcore_barrier()
    rows_per_tile = HALF // NS
    pltpu.sync_copy(
        acc_sh.at[pl.ds(t * rows_per_tile, rows_per_tile), :],
        acc_hbm.at[pl.ds(base_row + t * rows_per_tile, rows_per_tile), :])


_TCB = 5000


def _scale_matmul(h, deg, w):
    def body(h_ref, d_ref, w_ref, z_ref):
        sc = jax.lax.rsqrt(jnp.maximum(d_ref[...], 1.0))
        z_ref[...] = jnp.dot(h_ref[...] * sc, w_ref[...],
                             preferred_element_type=jnp.float32,
                             precision=jax.lax.Precision.HIGHEST)

    return pl.pallas_call(
        body,
        out_shape=jax.ShapeDtypeStruct((N, D), jnp.float32),
        grid=(N // _TCB,),
        in_specs=[pl.BlockSpec((_TCB, D), lambda i: (i, 0)),
                  pl.BlockSpec((_TCB, 1), lambda i: (i, 0)),
                  pl.BlockSpec((D, D), lambda i: (0, 0))],
        out_specs=pl.BlockSpec((_TCB, D), lambda i: (i, 0)),
    )(h, deg, w)


def _finalize(acc, deg, b):
    def body(a_ref, d_ref, b_ref, o_ref):
        sc = jax.lax.rsqrt(jnp.maximum(d_ref[...], 1.0))
        y = a_ref[...] * sc + b_ref[...]
        o_ref[...] = jnp.where(y > 0, y, jnp.exp(jnp.minimum(y, 0.0)) - 1.0)

    return pl.pallas_call(
        body,
        out_shape=jax.ShapeDtypeStruct((N, D), jnp.float32),
        grid=(N // _TCB,),
        in_specs=[pl.BlockSpec((_TCB, D), lambda i: (i, 0)),
                  pl.BlockSpec((_TCB, 1), lambda i: (i, 0)),
                  pl.BlockSpec((1, D), lambda i: (0, 0))],
        out_specs=pl.BlockSpec((_TCB, D), lambda i: (i, 0)),
    )(acc, deg, b)


def kernel(h_user, h_item, edge_index_user_to_item, edge_index_item_to_user, W, b):
    rowids = jnp.arange(2 * HR, dtype=jnp.int32)
    eall = jnp.concatenate([edge_index_user_to_item.reshape(-1),
                            edge_index_item_to_user.reshape(-1)])

    deg = _hist_kernel(eall, rowids)
    degf = deg.reshape(4, HR * 16)[:, :N]
    dout1, din1, dout2, din2 = (degf[i].reshape(N, 1) for i in range(4))

    z1 = _scale_matmul(h_user, dout1, W)
    z2 = _scale_matmul(h_item, dout2, W)

    acc1 = _agg_kernel(z1, edge_index_user_to_item)
    acc2 = _agg_kernel(z2, edge_index_item_to_user)

    out_item = _finalize(acc1, din1, b.reshape(1, D))
    out_user = _finalize(acc2, din2, b.reshape(1, D))
    return (out_user, out_item)
